# Initial kernel scaffold; baseline (speedup 1.0000x reference)
import jax
import jax.numpy as jnp
from jax.experimental import pallas as pl


def kernel(input_ids, context_vector, base_table, superposed_table):
    raise NotImplementedError("write your pallas kernel here")


# speedup vs baseline: 1.7080x; 1.7080x over previous
import jax
import jax.numpy as jnp
from jax import lax
from jax.experimental import pallas as pl
from jax.experimental.pallas import tpu as pltpu
from jax.experimental.pallas import tpu_sc as plsc

NC, NS, LANES = 2, 16, 16
NW = NC * NS
EMBED = 32
GATHER = 128
CHUNK = 1024
G = CHUNK // GATHER


def _sc_body(ids_hbm, ctx_hbm, base_hbm, sup_hbm, out_hbm,
             idx_v, ctx_v, brows, srows, sem):
    n_rows = out_hbm.shape[0]
    per_w = n_rows // NW
    n_chunks = per_w // CHUNK
    wid = lax.axis_index("s") * NC + lax.axis_index("c")

    def chunk_body(i, carry):
        row0 = wid * per_w + i * CHUNK
        pltpu.sync_copy(ids_hbm.at[pl.ds(wid * (per_w // GATHER) + i * G, G)], idx_v)
        pltpu.sync_copy(ctx_hbm.at[pl.ds(row0, CHUNK)], ctx_v)
        copies = []
        for g in range(G):
            dst = pl.ds(g * GATHER, GATHER)
            copies.append(pltpu.async_copy(base_hbm.at[idx_v.at[g]], brows.at[dst], sem))
            copies.append(pltpu.async_copy(sup_hbm.at[idx_v.at[g]], srows.at[dst], sem))
        for c in copies:
            c.wait()

        def blk_body(kb, rc):
            k0 = kb * LANES
            cvec = ctx_v[pl.ds(k0, LANES)]
            for j in range(LANES):
                cb = jnp.full((LANES,), cvec[j])
                for h in range(EMBED // LANES):
                    sl = (k0 + j, pl.ds(h * LANES, LANES))
                    plsc.addupdate(brows.at[sl], cb * srows[sl])
            return rc

        lax.fori_loop(0, CHUNK // LANES, blk_body, 0)
        pltpu.sync_copy(brows, out_hbm.at[pl.ds(row0, CHUNK)])
        return carry

    lax.fori_loop(0, n_chunks, chunk_body, 0)


def kernel(input_ids, context_vector, base_table, superposed_table):
    b, h = input_ids.shape
    n = b * h
    ids2d = input_ids.reshape(n // GATHER, GATHER).astype(jnp.int32)
    ctx = context_vector.reshape(n)
    mesh = plsc.VectorSubcoreMesh(core_axis_name="c", subcore_axis_name="s",
                                  num_cores=NC, num_subcores=NS)
    out = pl.kernel(
        _sc_body,
        out_type=jax.ShapeDtypeStruct((n, EMBED), jnp.float32),
        mesh=mesh,
        scratch_types=[
            pltpu.VMEM((G, GATHER), jnp.int32),
            pltpu.VMEM((CHUNK,), jnp.float32),
            pltpu.VMEM((CHUNK, EMBED), jnp.float32),
            pltpu.VMEM((CHUNK, EMBED), jnp.float32),
            pltpu.SemaphoreType.DMA,
        ],
        compiler_params=pltpu.CompilerParams(use_tc_tiling_on_sc=False),
    )(ids2d, ctx, base_table, superposed_table)
    return out.reshape(b, h, EMBED)
